# Initial kernel scaffold; baseline (speedup 1.0000x reference)
import functools

import jax
import jax.numpy as jnp
from jax import lax
from jax.experimental import pallas as pl
from jax.experimental.pallas import tpu as pltpu
from jax.experimental.pallas import tpu_sc as plsc

NU = 50000
NI = 50000
NTOT = NU + NI
HALF = 16

NC = 2
NS = 16
SUB = 128
BLK = 8
EPB = SUB * BLK
ZCH = 625


def _pad_edges(rows, cols, vals, row_offset=0):
    e = rows.shape[0]
    unit = NS * EPB
    ep = ((e + unit - 1) // unit) * unit
    pad = ep - e
    rows = jnp.pad(rows.astype(jnp.int32) + row_offset, (0, pad),
                   constant_values=row_offset)
    cols = jnp.pad(cols.astype(jnp.int32), (0, pad))
    vals = jnp.pad(vals, (0, pad))
    return (rows.reshape(-1, SUB), cols.reshape(-1, SUB),
            vals.reshape(-1, SUB), ep // unit)


def _to_half(x):
    return x.reshape(x.shape[0], NC, HALF).transpose(1, 0, 2)


def _from_half(x):
    return x.transpose(1, 0, 2).reshape(x.shape[1], NC * HALF)


def _body(nb_adj, nb_hyp, nb_ui,
          xt, hgt0, hgut0,
          a_c, a_r, a_v, h_c, h_r, h_v, u_c, u_r, u_v,
          final_t, hg_t, hgu_t, h1_t,
          acc, cidx, ridx, vbuf, gbuf, zbuf, cb0, cb1, sem):
    c = lax.axis_index("c")
    s = lax.axis_index("s")

    def zfill(i, _):
        zbuf[i, :] = jnp.zeros((HALF,), jnp.float32)
        return 0
    lax.fori_loop(0, ZCH, zfill, 0)

    def zero_acc(rows_per_tile, base=0):
        r0 = base + s * rows_per_tile
        for k in range(rows_per_tile // ZCH):
            pltpu.sync_copy(zbuf, acc.at[pl.ds(r0 + k * ZCH, ZCH)])

    def spmm(src_hbm, nb, cols, rows, vals):
        base_sub = s * nb * BLK

        def blk(b, _):
            sub0 = base_sub + b * BLK
            pltpu.sync_copy(cols.at[pl.ds(sub0, BLK)], cidx)
            pltpu.sync_copy(rows.at[pl.ds(sub0, BLK)], ridx)
            pltpu.sync_copy(vals.at[pl.ds(sub0, BLK)], vbuf)
            for j in range(BLK):
                pltpu.async_copy(src_hbm.at[cidx.at[j]], gbuf.at[j],
                                 sem).wait()

                def scale(e, _):
                    gbuf[j, e, :] = gbuf[j, e, :] * vbuf[j, e]
                    return 0
                lax.fori_loop(0, SUB, scale, 0)
                pltpu.sync_copy(gbuf.at[j], acc.at[ridx.at[j]], add=True)
            return 0
        lax.fori_loop(0, nb, blk, 0)

    def acc_to_hbm(dst, rows_per_tile, base=0):
        r0 = base + s * rows_per_tile
        pltpu.sync_copy(acc.at[pl.ds(r0, rows_per_tile)],
                        dst.at[pl.ds(r0 - base, rows_per_tile)])

    xt_c = xt.at[c]
    h1_c = h1_t.at[c]

    rpt = NTOT // NS
    zero_acc(rpt)
    plsc.subcore_barrier()
    spmm(xt_c, nb_adj, a_c, a_r, a_v)
    plsc.subcore_barrier()
    acc_to_hbm(h1_c, rpt)
    plsc.subcore_barrier()
    zero_acc(rpt)
    plsc.subcore_barrier()
    spmm(h1_c, nb_adj, a_c, a_r, a_v)
    plsc.subcore_barrier()

    third = jnp.float32(1.0 / 3.0)
    r0 = s * rpt
    for k in range(rpt // ZCH):
        rr = r0 + k * ZCH
        pltpu.sync_copy(xt_c.at[pl.ds(rr, ZCH)], cb0)
        pltpu.sync_copy(h1_c.at[pl.ds(rr, ZCH)], cb1)

        def add1(i, _):
            cb0[i, :] = cb0[i, :] + cb1[i, :]
            return 0
        lax.fori_loop(0, ZCH, add1, 0)
        pltpu.sync_copy(acc.at[pl.ds(rr, ZCH)], cb1)

        def add2(i, _):
            cb0[i, :] = (cb0[i, :] + cb1[i, :]) * third
            return 0
        lax.fori_loop(0, ZCH, add2, 0)
        pltpu.sync_copy(cb0, final_t.at[c].at[pl.ds(rr, ZCH)])

    rpt_i = NI // NS
    plsc.subcore_barrier()
    zero_acc(rpt_i)
    plsc.subcore_barrier()
    spmm(hgt0.at[c], nb_hyp, h_c, h_r, h_v)
    plsc.subcore_barrier()
    acc_to_hbm(h1_c, rpt_i)
    plsc.subcore_barrier()
    zero_acc(rpt_i)
    plsc.subcore_barrier()
    spmm(h1_c, nb_hyp, h_c, h_r, h_v)
    plsc.subcore_barrier()
    acc_to_hbm(hg_t.at[c], rpt_i)

    pltpu.sync_copy(hgut0.at[c].at[pl.ds(s * rpt_i, rpt_i)],
                    acc.at[pl.ds(NI + s * rpt_i, rpt_i)])
    plsc.subcore_barrier()
    spmm(hg_t.at[c], nb_ui, u_c, u_r, u_v)
    plsc.subcore_barrier()
    acc_to_hbm(hgu_t.at[c], rpt_i, base=NI)


def kernel(user_embeds, item_embeds, hg_user_embeds, hg_item_embeds,
           adj_val, hyper_val, ui_val, adj_idx, hyper_idx, ui_idx):
    xt = _to_half(jnp.concatenate([user_embeds, item_embeds], axis=0))
    hgt0 = _to_half(hg_item_embeds)
    hgut0 = _to_half(hg_user_embeds)

    a_r, a_c, a_v, nb_adj = _pad_edges(adj_idx[0], adj_idx[1], adj_val)
    h_r, h_c, h_v, nb_hyp = _pad_edges(hyper_idx[0], hyper_idx[1], hyper_val)
    u_r, u_c, u_v, nb_ui = _pad_edges(ui_idx[0], ui_idx[1], ui_val,
                                      row_offset=NI)

    mesh = plsc.VectorSubcoreMesh(core_axis_name="c", subcore_axis_name="s")
    f32 = jnp.float32
    out_type = (
        jax.ShapeDtypeStruct((NC, NTOT, HALF), f32),
        jax.ShapeDtypeStruct((NC, NI, HALF), f32),
        jax.ShapeDtypeStruct((NC, NU, HALF), f32),
        jax.ShapeDtypeStruct((NC, NTOT, HALF), f32),
    )
    scratch = [
        pltpu.VMEM_SHARED((NTOT, HALF), f32),
        pltpu.VMEM((BLK, SUB), jnp.int32),
        pltpu.VMEM((BLK, SUB), jnp.int32),
        pltpu.VMEM((BLK, SUB), f32),
        pltpu.VMEM((BLK, SUB, HALF), f32),
        pltpu.VMEM((ZCH, HALF), f32),
        pltpu.VMEM((ZCH, HALF), f32),
        pltpu.VMEM((ZCH, HALF), f32),
        pltpu.SemaphoreType.DMA,
    ]
    run = pl.kernel(
        functools.partial(_body, nb_adj, nb_hyp, nb_ui),
        out_type=out_type,
        mesh=mesh,
        scratch_types=scratch,
    )
    final_t, hg_t, hgu_t, _ = run(xt, hgt0, hgut0,
                                  a_c, a_r, a_v, h_c, h_r, h_v,
                                  u_c, u_r, u_v)
    final = _from_half(final_t)
    return (final[:NU], final[NU:], _from_half(hgu_t), _from_half(hg_t))


# speedup vs baseline: 5.9940x; 5.9940x over previous
import functools

import jax
import jax.numpy as jnp
from jax import lax
from jax.experimental import pallas as pl
from jax.experimental.pallas import tpu as pltpu
from jax.experimental.pallas import tpu_sc as plsc

NU = 50000
NI = 50000
NTOT = NU + NI
HALF = 16

NC = 2
NS = 16
SUB = 128
BLK = 8
NG = 2

NI_P = 50048
NTOT_P = 100096
RPT = NTOT_P // NS
RPT_I = NI_P // NS
CH = 512


def _chunks(total):
    out = []
    o = 0
    while o < total:
        sz = min(CH, total - o)
        out.append((o, sz))
        o += sz
    return out


def _pad_edges(rows, cols, vals, row_offset=0):
    e = rows.shape[0]
    unit = NS * BLK * SUB
    ep = ((e + unit - 1) // unit) * unit
    pad = ep - e
    rows = jnp.pad(rows.astype(jnp.int32) + row_offset, (0, pad),
                   constant_values=row_offset)
    cols = jnp.pad(cols.astype(jnp.int32), (0, pad))
    vals = jnp.pad(vals, (0, pad))
    return (rows.reshape(-1, SUB), cols.reshape(-1, SUB),
            vals.reshape(-1, SUB), ep // unit)


def _to_half(x, vpad):
    x = jnp.pad(x, ((0, vpad - x.shape[0]), (0, 0)))
    return x.reshape(vpad, NC, HALF).transpose(1, 0, 2)


def _from_half(x, v):
    return x.transpose(1, 0, 2).reshape(x.shape[1], NC * HALF)[:v]


def _body(nb_adj, nb_hyp, nb_ui,
          xt, hgt0, hgut0,
          a_c, a_r, a_v, h_c, h_r, h_v, u_c, u_r, u_v,
          final_t, hg_t, hgu_t, h1_t,
          acc, cidx, ridx, vbuf, gbuf, cb0, cb1, sem):
    c = lax.axis_index("c")
    s = lax.axis_index("s")

    def fill_zero():
        def zf(i, _):
            cb0[i, :] = jnp.zeros((HALF,), jnp.float32)
            return 0
        lax.fori_loop(0, CH, zf, 0)

    def zero_acc(rows_per_tile):
        r0 = s * rows_per_tile
        for o, sz in _chunks(rows_per_tile):
            pltpu.sync_copy(cb0.at[pl.ds(0, sz)], acc.at[pl.ds(r0 + o, sz)])

    def spmm(src_hbm, nb, cols, rows, vals):
        base_sub = s * nb * BLK

        def blk(b, _):
            sub0 = base_sub + b * BLK
            pltpu.sync_copy(cols.at[pl.ds(sub0, BLK)], cidx)
            pltpu.sync_copy(rows.at[pl.ds(sub0, BLK)], ridx)
            pltpu.sync_copy(vals.at[pl.ds(sub0, BLK)], vbuf)
            for j in range(BLK):
                g = gbuf.at[j % NG]
                pltpu.async_copy(src_hbm.at[cidx.at[j]], g, sem).wait()

                def scale16(gr, _):
                    e0 = gr * HALF
                    vv = vbuf[j, pl.ds(e0, HALF)]
                    for i in range(HALF):
                        g[e0 + i, :] = g[e0 + i, :] * vv[i]
                    return 0
                lax.fori_loop(0, SUB // HALF, scale16, 0)
                pltpu.sync_copy(g, acc.at[ridx.at[j]], add=True)
            return 0
        lax.fori_loop(0, nb, blk, 0)

    def acc_to_hbm(dst, rows_per_tile, base=0):
        r0 = base + s * rows_per_tile
        pltpu.sync_copy(acc.at[pl.ds(r0, rows_per_tile)],
                        dst.at[pl.ds(r0 - base, rows_per_tile)])

    xt_c = xt.at[c]
    h1_c = h1_t.at[c]

    fill_zero()
    zero_acc(RPT)
    plsc.subcore_barrier()
    spmm(xt_c, nb_adj, a_c, a_r, a_v)
    plsc.subcore_barrier()
    acc_to_hbm(h1_c, RPT)
    plsc.subcore_barrier()
    zero_acc(RPT)
    plsc.subcore_barrier()
    spmm(h1_c, nb_adj, a_c, a_r, a_v)
    plsc.subcore_barrier()

    third = jnp.float32(1.0 / 3.0)
    r0 = s * RPT
    for o, sz in _chunks(RPT):
        rr = r0 + o
        pltpu.sync_copy(xt_c.at[pl.ds(rr, sz)], cb0.at[pl.ds(0, sz)])
        pltpu.sync_copy(h1_c.at[pl.ds(rr, sz)], cb1.at[pl.ds(0, sz)])

        def add1(i, _):
            cb0[i, :] = cb0[i, :] + cb1[i, :]
            return 0
        lax.fori_loop(0, sz, add1, 0)
        pltpu.sync_copy(acc.at[pl.ds(rr, sz)], cb1.at[pl.ds(0, sz)])

        def add2(i, _):
            cb0[i, :] = (cb0[i, :] + cb1[i, :]) * third
            return 0
        lax.fori_loop(0, sz, add2, 0)
        pltpu.sync_copy(cb0.at[pl.ds(0, sz)],
                        final_t.at[c].at[pl.ds(rr, sz)])

    plsc.subcore_barrier()
    fill_zero()
    zero_acc(RPT_I)
    plsc.subcore_barrier()
    spmm(hgt0.at[c], nb_hyp, h_c, h_r, h_v)
    plsc.subcore_barrier()
    acc_to_hbm(h1_c, RPT_I)
    plsc.subcore_barrier()
    zero_acc(RPT_I)
    plsc.subcore_barrier()
    spmm(h1_c, nb_hyp, h_c, h_r, h_v)
    plsc.subcore_barrier()
    acc_to_hbm(hg_t.at[c], RPT_I)

    pltpu.sync_copy(hgut0.at[c].at[pl.ds(s * RPT_I, RPT_I)],
                    acc.at[pl.ds(NI_P + s * RPT_I, RPT_I)])
    plsc.subcore_barrier()
    spmm(hg_t.at[c], nb_ui, u_c, u_r, u_v)
    plsc.subcore_barrier()
    acc_to_hbm(hgu_t.at[c], RPT_I, base=NI_P)


def kernel(user_embeds, item_embeds, hg_user_embeds, hg_item_embeds,
           adj_val, hyper_val, ui_val, adj_idx, hyper_idx, ui_idx):
    xt = _to_half(jnp.concatenate([user_embeds, item_embeds], axis=0), NTOT_P)
    hgt0 = _to_half(hg_item_embeds, NI_P)
    hgut0 = _to_half(hg_user_embeds, NI_P)

    a_r, a_c, a_v, nb_adj = _pad_edges(adj_idx[0], adj_idx[1], adj_val)
    h_r, h_c, h_v, nb_hyp = _pad_edges(hyper_idx[0], hyper_idx[1], hyper_val)
    u_r, u_c, u_v, nb_ui = _pad_edges(ui_idx[0], ui_idx[1], ui_val,
                                      row_offset=NI_P)

    mesh = plsc.VectorSubcoreMesh(core_axis_name="c", subcore_axis_name="s")
    f32 = jnp.float32
    out_type = (
        jax.ShapeDtypeStruct((NC, NTOT_P, HALF), f32),
        jax.ShapeDtypeStruct((NC, NI_P, HALF), f32),
        jax.ShapeDtypeStruct((NC, NI_P, HALF), f32),
        jax.ShapeDtypeStruct((NC, NTOT_P, HALF), f32),
    )
    scratch = [
        pltpu.VMEM_SHARED((NTOT_P, HALF), f32),
        pltpu.VMEM((BLK, SUB), jnp.int32),
        pltpu.VMEM((BLK, SUB), jnp.int32),
        pltpu.VMEM((BLK, SUB), f32),
        pltpu.VMEM((NG, SUB, HALF), f32),
        pltpu.VMEM((CH, HALF), f32),
        pltpu.VMEM((CH, HALF), f32),
        pltpu.SemaphoreType.DMA,
    ]
    run = pl.kernel(
        functools.partial(_body, nb_adj, nb_hyp, nb_ui),
        out_type=out_type,
        mesh=mesh,
        scratch_types=scratch,
        compiler_params=pltpu.CompilerParams(use_tc_tiling_on_sc=False),
    )
    final_t, hg_t, hgu_t, _ = run(xt, hgt0, hgut0,
                                  a_c, a_r, a_v, h_c, h_r, h_v,
                                  u_c, u_r, u_v)
    final = _from_half(final_t, NTOT)
    return (final[:NU], final[NU:],
            _from_half(hgu_t, NU), _from_half(hg_t, NI))
